# Initial kernel scaffold; baseline (speedup 1.0000x reference)
import jax
import jax.numpy as jnp
from jax.experimental import pallas as pl


def kernel(symbols, operations, sym_table, op_table, W, b):
    raise NotImplementedError("write your pallas kernel here")


# speedup vs baseline: 2.7358x; 2.7358x over previous
import functools

import jax
import jax.numpy as jnp
from jax import lax
from jax.experimental import pallas as pl
from jax.experimental.pallas import tpu as pltpu
from jax.experimental.pallas import tpu_sc as plsc

L = 16
NC, NS = 2, 16
NW = NC * NS

D = 512
CHUNK = 64


def _project_kernel(sym_ref, op_ref, ws_ref, wo_ref, b_ref, symp_ref, opp_ref):
    dn = (((1,), (1,)), ((), ()))
    symp_ref[...] = lax.dot_general(sym_ref[...], ws_ref[...], dn,
                                    preferred_element_type=jnp.float32)
    opp_ref[...] = lax.dot_general(op_ref[...], wo_ref[...], dn,
                                   preferred_element_type=jnp.float32) + b_ref[...]


def _project_tables(sym_table, op_table, Ws, Wo, b2d):
    return pl.pallas_call(
        _project_kernel,
        out_shape=(
            jax.ShapeDtypeStruct((1000, D), jnp.float32),
            jax.ShapeDtypeStruct((100, D), jnp.float32),
        ),
    )(sym_table, op_table, Ws, Wo, b2d)


def _gather_add(sym_idx, op_idx, sym_proj, op_proj, n_tokens):
    tok_per_w = n_tokens // NW
    n_chunks = tok_per_w // CHUNK
    mesh = plsc.VectorSubcoreMesh(core_axis_name="c", subcore_axis_name="s")

    @functools.partial(
        pl.kernel,
        mesh=mesh,
        out_type=jax.ShapeDtypeStruct((n_tokens, D), jnp.float32),
        scratch_types=[
            pltpu.VMEM((CHUNK,), jnp.int32),
            pltpu.VMEM((CHUNK,), jnp.int32),
            pltpu.VMEM((CHUNK, D), jnp.float32),
            pltpu.VMEM((CHUNK, D), jnp.float32),
            pltpu.SemaphoreType.DMA,
            pltpu.SemaphoreType.DMA,
        ],
    )
    def k(sym_idx_hbm, op_idx_hbm, symp_hbm, opp_hbm, out_hbm,
          idx_s, idx_o, buf_s, buf_o, sem_s, sem_o):
        wid = lax.axis_index("s") * NC + lax.axis_index("c")
        w_base = wid * tok_per_w

        def chunk_body(c, carry):
            base = w_base + c * CHUNK
            pltpu.sync_copy(sym_idx_hbm.at[pl.ds(base, CHUNK)], idx_s)
            pltpu.sync_copy(op_idx_hbm.at[pl.ds(base, CHUNK)], idx_o)
            cp_s = pltpu.async_copy(symp_hbm.at[idx_s], buf_s, sem_s)
            cp_o = pltpu.async_copy(opp_hbm.at[idx_o], buf_o, sem_o)
            cp_s.wait()
            cp_o.wait()

            def add_row(t, carry2):
                for dd in range(D // L):
                    sl = pl.ds(dd * L, L)
                    plsc.addupdate(buf_s.at[t, sl], buf_o[t, sl])
                return carry2

            lax.fori_loop(0, CHUNK, add_row, 0, unroll=2)
            pltpu.sync_copy(buf_s, out_hbm.at[pl.ds(base, CHUNK)])
            return carry

        lax.fori_loop(0, n_chunks, chunk_body, 0)

    return k(sym_idx, op_idx, sym_proj, op_proj)


def kernel(symbols, operations, sym_table, op_table, W, b):
    B, Lseq = symbols.shape
    n_tokens = B * Lseq
    sym_proj, op_proj = _project_tables(
        sym_table, op_table, W[:, :D], W[:, D:], b.reshape(1, D))
    flat_out = _gather_add(
        symbols.reshape(n_tokens).astype(jnp.int32),
        operations.reshape(n_tokens).astype(jnp.int32),
        sym_proj, op_proj, n_tokens)
    return flat_out.reshape(B, Lseq, D)


# speedup vs baseline: 3.6403x; 1.3306x over previous
import functools

import jax
import jax.numpy as jnp
from jax import lax
from jax.experimental import pallas as pl
from jax.experimental.pallas import tpu as pltpu
from jax.experimental.pallas import tpu_sc as plsc

L = 16
NC, NS = 2, 16
NW = NC * NS

D = 512
CHUNK = 40


def _project_kernel(sym_ref, op_ref, ws_ref, wo_ref, b_ref, symp_ref, opp_ref):
    dn = (((1,), (1,)), ((), ()))
    symp_ref[...] = lax.dot_general(sym_ref[...], ws_ref[...], dn,
                                    preferred_element_type=jnp.float32)
    opp_ref[...] = lax.dot_general(op_ref[...], wo_ref[...], dn,
                                   preferred_element_type=jnp.float32) + b_ref[...]


def _project_tables(sym_table, op_table, Ws, Wo, b2d):
    return pl.pallas_call(
        _project_kernel,
        out_shape=(
            jax.ShapeDtypeStruct((1000, D), jnp.float32),
            jax.ShapeDtypeStruct((100, D), jnp.float32),
        ),
    )(sym_table, op_table, Ws, Wo, b2d)


def _gather_add(sym_idx, op_idx, sym_proj, op_proj, n_tokens):
    tok_per_w = n_tokens // NW
    n_chunks = tok_per_w // CHUNK
    n_pairs = n_chunks // 2
    mesh = plsc.VectorSubcoreMesh(core_axis_name="c", subcore_axis_name="s")

    @functools.partial(
        pl.kernel,
        mesh=mesh,
        out_type=jax.ShapeDtypeStruct((n_tokens, D), jnp.float32),
        scratch_types=[
            pltpu.VMEM((3, CHUNK), jnp.int32),
            pltpu.VMEM((3, CHUNK), jnp.int32),
            pltpu.VMEM((CHUNK, D), jnp.float32),
            pltpu.VMEM((CHUNK, D), jnp.float32),
            pltpu.VMEM((CHUNK, D), jnp.float32),
            pltpu.VMEM((CHUNK, D), jnp.float32),
            pltpu.SemaphoreType.DMA,
            pltpu.SemaphoreType.DMA,
            pltpu.SemaphoreType.DMA,
            pltpu.SemaphoreType.DMA,
            pltpu.SemaphoreType.DMA,
        ],
    )
    def k(sym_idx_hbm, op_idx_hbm, symp_hbm, opp_hbm, out_hbm,
          idx_s3, idx_o3, buf_s0, buf_s1, buf_o0, buf_o1,
          sem_i, sem_g0, sem_g1, sem_w0, sem_w1):
        wid = lax.axis_index("s") * NC + lax.axis_index("c")
        w_base = wid * tok_per_w
        bufs_s = (buf_s0, buf_s1)
        bufs_o = (buf_o0, buf_o1)
        sems_g = (sem_g0, sem_g1)
        sems_w = (sem_w0, sem_w1)

        def fire_idx(c):
            j = lax.rem(c, 3)
            base = w_base + c * CHUNK
            pltpu.async_copy(sym_idx_hbm.at[pl.ds(base, CHUNK)], idx_s3.at[j], sem_i)
            pltpu.async_copy(op_idx_hbm.at[pl.ds(base, CHUNK)], idx_o3.at[j], sem_i)

        def drain_idx():
            pltpu.make_async_copy(
                sym_idx_hbm.at[pl.ds(0, CHUNK)], idx_s3.at[0], sem_i).wait()
            pltpu.make_async_copy(
                op_idx_hbm.at[pl.ds(0, CHUNK)], idx_o3.at[0], sem_i).wait()

        def fire_gathers(c, b):
            j = lax.rem(c, 3)
            pltpu.async_copy(symp_hbm.at[idx_s3.at[j]], bufs_s[b], sems_g[b])
            pltpu.async_copy(opp_hbm.at[idx_o3.at[j]], bufs_o[b], sems_g[b])

        def drain_gathers(b):
            dummy = out_hbm.at[pl.ds(0, CHUNK)]
            pltpu.make_async_copy(dummy, bufs_s[b], sems_g[b]).wait()
            pltpu.make_async_copy(dummy, bufs_o[b], sems_g[b]).wait()

        def drain_writeout(b):
            pltpu.make_async_copy(
                bufs_s[b], out_hbm.at[pl.ds(0, CHUNK)], sems_w[b]).wait()

        fire_idx(0)
        fire_idx(1)
        drain_idx()
        fire_gathers(0, 0)

        def pair_body(p, carry):
            for b in range(2):
                c = 2 * p + b
                b1 = 1 - b

                @pl.when(c >= 1)
                def _():
                    drain_writeout(b1)

                @pl.when(c + 1 < n_chunks)
                def _():
                    drain_idx()
                    fire_gathers(c + 1, b1)

                @pl.when(c + 2 < n_chunks)
                def _():
                    fire_idx(c + 2)

                drain_gathers(b)

                def add_row(t, carry2):
                    for dd in range(D // L):
                        sl = pl.ds(dd * L, L)
                        plsc.addupdate(bufs_s[b].at[t, sl], bufs_o[b][t, sl])
                    return carry2

                lax.fori_loop(0, CHUNK, add_row, 0, unroll=2)

                base = w_base + c * CHUNK
                pltpu.async_copy(bufs_s[b], out_hbm.at[pl.ds(base, CHUNK)],
                                 sems_w[b])
            return carry

        lax.fori_loop(0, n_pairs, pair_body, 0)
        drain_writeout((n_chunks - 1) % 2)

    return k(sym_idx, op_idx, sym_proj, op_proj)


def kernel(symbols, operations, sym_table, op_table, W, b):
    B, Lseq = symbols.shape
    n_tokens = B * Lseq
    sym_proj, op_proj = _project_tables(
        sym_table, op_table, W[:, :D], W[:, D:], b.reshape(1, D))
    flat_out = _gather_add(
        symbols.reshape(n_tokens).astype(jnp.int32),
        operations.reshape(n_tokens).astype(jnp.int32),
        sym_proj, op_proj, n_tokens)
    return flat_out.reshape(B, Lseq, D)
